# Initial kernel scaffold; baseline (speedup 1.0000x reference)
import functools

import jax
import jax.numpy as jnp
from jax import lax
from jax.experimental import pallas as pl
from jax.experimental.pallas import tpu as pltpu
from jax.experimental.pallas import tpu_sc as plsc

N = 10000
D = 128
E = 320000
EL = 200000

NC = 2
NS = 16
NW = NC * NS

NPAD = 10016
EPW = 10240
EPAD = NW * EPW
SEG_K = 512
SEG_CHUNKS = EPW // SEG_K

ELW = 6656
ELPAD = NW * ELW
CLS_K = 256
CLS_CHUNKS = ELW // CLS_K

ROWS_PER_TILE = N // NS


def _seg_sum_body(with_cnt, *refs):
    if with_cnt:
        (src_hbm, dst_hbm, x_hbm, z128_hbm, z16_hbm, ones_hbm,
         agg_out, cnt_out,
         sidx_v, didx_v, rows_v, ones_v, agg_sp, cnt_sp, sem) = refs
    else:
        (src_hbm, dst_hbm, x_hbm, z128_hbm,
         agg_out,
         sidx_v, didx_v, rows_v, agg_sp, sem) = refs

    c = lax.axis_index("c")
    s = lax.axis_index("s")
    w = s * NC + c

    r0 = s * ROWS_PER_TILE
    pltpu.sync_copy(z128_hbm.at[pl.ds(r0, ROWS_PER_TILE)],
                    agg_sp.at[pl.ds(r0, ROWS_PER_TILE)])
    if with_cnt:
        pltpu.sync_copy(z16_hbm.at[pl.ds(r0, ROWS_PER_TILE)],
                        cnt_sp.at[pl.ds(r0, ROWS_PER_TILE)])
        pltpu.sync_copy(ones_hbm, ones_v)
    plsc.subcore_barrier()

    row_base = w * (EPW // 128)

    def chunk(i, carry):
        off = row_base + i * (SEG_K // 128)
        pltpu.sync_copy(src_hbm.at[pl.ds(off, SEG_K // 128)], sidx_v)
        pltpu.sync_copy(dst_hbm.at[pl.ds(off, SEG_K // 128)], didx_v)
        for j in range(SEG_K // 128):
            pltpu.async_copy(x_hbm.at[sidx_v.at[j]],
                             rows_v.at[pl.ds(j * 128, 128)], sem).wait()
            pltpu.sync_copy(rows_v.at[pl.ds(j * 128, 128)],
                            agg_sp.at[didx_v.at[j]], add=True)
            if with_cnt:
                pltpu.sync_copy(ones_v.at[pl.ds(j * 128, 128)],
                                cnt_sp.at[didx_v.at[j]], add=True)
        return carry

    lax.fori_loop(0, SEG_CHUNKS, chunk, 0)
    plsc.subcore_barrier()

    pltpu.sync_copy(agg_sp.at[pl.ds(r0, ROWS_PER_TILE)],
                    agg_out.at[c, pl.ds(r0, ROWS_PER_TILE)])
    if with_cnt:
        pltpu.sync_copy(cnt_sp.at[pl.ds(r0, ROWS_PER_TILE)],
                        cnt_out.at[c, pl.ds(r0, ROWS_PER_TILE)])


def _make_seg_sum(with_cnt):
    mesh = plsc.VectorSubcoreMesh(core_axis_name="c", subcore_axis_name="s")
    out_type = [jax.ShapeDtypeStruct((NC, N, D), jnp.float32)]
    scratch = [
        pltpu.VMEM((SEG_K // 128, 128), jnp.int32),
        pltpu.VMEM((SEG_K // 128, 128), jnp.int32),
        pltpu.VMEM((SEG_K, D), jnp.float32),
    ]
    if with_cnt:
        out_type.append(jax.ShapeDtypeStruct((NC, N, 16), jnp.float32))
        scratch.append(pltpu.VMEM((SEG_K, 16), jnp.float32))
        scratch.append(pltpu.VMEM_SHARED((NPAD, D), jnp.float32))
        scratch.append(pltpu.VMEM_SHARED((NPAD, 16), jnp.float32))
    else:
        scratch.append(pltpu.VMEM_SHARED((NPAD, D), jnp.float32))
    scratch.append(pltpu.SemaphoreType.DMA)
    return pl.kernel(
        functools.partial(_seg_sum_body, with_cnt),
        out_type=out_type,
        mesh=mesh,
        scratch_types=scratch,
    )


def _cls_body(h_hbm, e0_hbm, e1_hbm, pred_out,
              i0_v, i1_v, a_v, b_v, st_v, out_v, sem):
    c = lax.axis_index("c")
    s = lax.axis_index("s")
    w = s * NC + c
    row_base = w * (ELW // 128)
    lanes = lax.iota(jnp.int32, 16)

    def chunk(t, carry):
        off = row_base + t * (CLS_K // 128)
        pltpu.sync_copy(e0_hbm.at[pl.ds(off, CLS_K // 128)], i0_v)
        pltpu.sync_copy(e1_hbm.at[pl.ds(off, CLS_K // 128)], i1_v)
        waits = []
        for j in range(CLS_K // 128):
            waits.append(pltpu.async_copy(
                h_hbm.at[i0_v.at[j]], a_v.at[pl.ds(j * 128, 128)], sem))
            waits.append(pltpu.async_copy(
                h_hbm.at[i1_v.at[j]], b_v.at[pl.ds(j * 128, 128)], sem))
        for d in waits:
            d.wait()

        def row(r, carry2):
            acc = a_v[r, pl.ds(0, 16)] * b_v[r, pl.ds(0, 16)]
            for j in range(1, D // 16):
                acc = acc + a_v[r, pl.ds(j * 16, 16)] * b_v[r, pl.ds(j * 16, 16)]
            plsc.store_scatter(st_v, [lanes, jnp.full((16,), r, jnp.int32)], acc)
            return carry2

        lax.fori_loop(0, CLS_K, row, 0)

        def grp(g, carry2):
            tot = st_v[0, pl.ds(g * 16, 16)]
            for l in range(1, 16):
                tot = tot + st_v[l, pl.ds(g * 16, 16)]
            out_v[pl.ds(g * 16, 16)] = tot
            return carry2

        lax.fori_loop(0, CLS_K // 16, grp, 0)
        pltpu.sync_copy(out_v, pred_out.at[pl.ds(w * ELW + t * CLS_K, CLS_K)])
        return carry

    lax.fori_loop(0, CLS_CHUNKS, chunk, 0)


_cls_kernel = pl.kernel(
    _cls_body,
    out_type=jax.ShapeDtypeStruct((ELPAD,), jnp.float32),
    mesh=plsc.VectorSubcoreMesh(core_axis_name="c", subcore_axis_name="s"),
    scratch_types=[
        pltpu.VMEM((CLS_K // 128, 128), jnp.int32),
        pltpu.VMEM((CLS_K // 128, 128), jnp.int32),
        pltpu.VMEM((CLS_K, D), jnp.float32),
        pltpu.VMEM((CLS_K, D), jnp.float32),
        pltpu.VMEM((16, CLS_K), jnp.float32),
        pltpu.VMEM((CLS_K,), jnp.float32),
        pltpu.SemaphoreType.DMA,
    ],
)


def _tc_body(relu, agg_ref, cnt_ref, x_ref, wl_ref, wr_ref, bl_ref, out_ref):
    aggs = agg_ref[0] + agg_ref[1]
    cnt = cnt_ref[0, :, 0:1] + cnt_ref[1, :, 0:1]
    mean = aggs / jnp.maximum(cnt, 1.0)
    h = lax.dot_general(mean, wl_ref[...], (((1,), (1,)), ((), ())),
                        preferred_element_type=jnp.float32)
    h = h + bl_ref[...]
    h = h + lax.dot_general(x_ref[...], wr_ref[...], (((1,), (1,)), ((), ())),
                            preferred_element_type=jnp.float32)
    if relu:
        h = jnp.maximum(h, 0.0)
    out_ref[...] = h


def _tc_layer(relu, agg, cnt, x, wl, wr, bl):
    R = 1000
    grid = (N // R,)
    return pl.pallas_call(
        functools.partial(_tc_body, relu),
        grid=grid,
        in_specs=[
            pl.BlockSpec((NC, R, D), lambda i: (0, i, 0)),
            pl.BlockSpec((NC, R, 16), lambda i: (0, i, 0)),
            pl.BlockSpec((R, D), lambda i: (i, 0)),
            pl.BlockSpec((D, D), lambda i: (0, 0)),
            pl.BlockSpec((D, D), lambda i: (0, 0)),
            pl.BlockSpec((1, D), lambda i: (0, 0)),
        ],
        out_specs=pl.BlockSpec((R, D), lambda i: (i, 0)),
        out_shape=jax.ShapeDtypeStruct((N, D), jnp.float32),
    )(agg, cnt, x, wl, wr, bl)


_seg_sum_cnt = _make_seg_sum(True)
_seg_sum = _make_seg_sum(False)


def kernel(x, edge_index, edge_label_index, Wl1, bl1, Wr1, Wl2, bl2, Wr2):
    ei = edge_index.astype(jnp.int32)
    eli = edge_label_index.astype(jnp.int32)

    pad = EPAD - E
    src = jnp.concatenate([ei[0], jnp.zeros((pad,), jnp.int32)])
    dst = jnp.concatenate([ei[1], jnp.full((pad,), N, jnp.int32)])
    src2 = src.reshape(EPAD // 128, 128)
    dst2 = dst.reshape(EPAD // 128, 128)

    z128 = jnp.zeros((N, D), jnp.float32)
    z16 = jnp.zeros((N, 16), jnp.float32)
    ones = jnp.ones((SEG_K, 16), jnp.float32)

    agg1, cnt = _seg_sum_cnt(src2, dst2, x, z128, z16, ones)
    h1 = _tc_layer(True, agg1, cnt, x,
                   Wl1, Wr1, bl1.reshape(1, D))
    agg2 = _seg_sum(src2, dst2, h1, z128)
    h2 = _tc_layer(False, agg2, cnt, h1,
                   Wl2, Wr2, bl2.reshape(1, D))

    lpad = ELPAD - EL
    e0 = jnp.concatenate([eli[0], jnp.zeros((lpad,), jnp.int32)])
    e1 = jnp.concatenate([eli[1], jnp.zeros((lpad,), jnp.int32)])
    e0r = e0.reshape(ELPAD // 128, 128)
    e1r = e1.reshape(ELPAD // 128, 128)
    pred = _cls_kernel(h2, e0r, e1r)
    return pred[:EL]


# speedup vs baseline: 2.8748x; 2.8748x over previous
import functools

import jax
import jax.numpy as jnp
from jax import lax
from jax.experimental import pallas as pl
from jax.experimental.pallas import tpu as pltpu
from jax.experimental.pallas import tpu_sc as plsc

N = 10000
D = 128
E = 320000
EL = 200000

NC = 2
NS = 16
NW = NC * NS

NR = 10240
STRIPE = NR // NS
EPW = 10240
EPAD = NW * EPW

ELW = 6272
ELPAD = NW * ELW

_MESH = dict(core_axis_name="c", subcore_axis_name="s",
             num_cores=NC, num_subcores=NS)


def _seg_sum_body(with_gather, *refs):
    if with_gather:
        (src_hbm, dst_hbm, x_hbm, z_hbm, agg_out,
         sidx_w, didx_w, rows_w, agg_sp, sem) = refs
    else:
        (dst_hbm, z_hbm, agg_out,
         didx_w, rows_w, agg_sp, sem) = refs

    c = lax.axis_index("c")
    s = lax.axis_index("s")
    w = s * NC + c

    r0 = s * STRIPE
    for q in range(STRIPE // 128):
        pltpu.sync_copy(z_hbm.at[pl.ds(r0 + q * 128, 128)], rows_w)
        pltpu.sync_copy(rows_w, agg_sp.at[pl.ds(r0 + q * 128, 128)])
    plsc.subcore_barrier()

    base = w * EPW
    if not with_gather:
        ones = jnp.ones((16,), jnp.float32)

        def fill(i, carry):
            for j in range(D // 16):
                rows_w[i, pl.ds(j * 16, 16)] = ones
            return carry

        lax.fori_loop(0, 128, fill, 0)

    def burst(i, carry):
        off = base + i * 128
        pltpu.sync_copy(dst_hbm.at[pl.ds(off, 128)], didx_w)
        if with_gather:
            pltpu.sync_copy(src_hbm.at[pl.ds(off, 128)], sidx_w)
            pltpu.async_copy(x_hbm.at[sidx_w], rows_w, sem).wait()
        pltpu.sync_copy(rows_w, agg_sp.at[didx_w], add=True)
        return carry

    lax.fori_loop(0, EPW // 128, burst, 0)
    plsc.subcore_barrier()

    out_base = c * NR + r0
    for q in range(STRIPE // 128):
        pltpu.sync_copy(agg_sp.at[pl.ds(r0 + q * 128, 128)], rows_w)
        pltpu.sync_copy(rows_w, agg_out.at[pl.ds(out_base + q * 128, 128)])


def _make_seg_sum(with_gather):
    scratch = [pltpu.VMEM((128,), jnp.int32)]
    if with_gather:
        scratch.append(pltpu.VMEM((128,), jnp.int32))
    scratch += [
        pltpu.VMEM((128, D), jnp.float32),
        pltpu.VMEM_SHARED((NR, D), jnp.float32),
        pltpu.SemaphoreType.DMA,
    ]
    if with_gather:
        scratch = [pltpu.VMEM((128,), jnp.int32)] + scratch[:1] + scratch[1:]
        scratch = [
            pltpu.VMEM((128,), jnp.int32),
            pltpu.VMEM((128,), jnp.int32),
            pltpu.VMEM((128, D), jnp.float32),
            pltpu.VMEM_SHARED((NR, D), jnp.float32),
            pltpu.SemaphoreType.DMA,
        ]
    else:
        scratch = [
            pltpu.VMEM((128,), jnp.int32),
            pltpu.VMEM((128, D), jnp.float32),
            pltpu.VMEM_SHARED((NR, D), jnp.float32),
            pltpu.SemaphoreType.DMA,
        ]
    return pl.kernel(
        functools.partial(_seg_sum_body, with_gather),
        out_type=jax.ShapeDtypeStruct((NC * NR, D), jnp.float32),
        mesh=plsc.VectorSubcoreMesh(**_MESH),
        scratch_types=scratch,
    )


def _cls_body(h_hbm, e0_hbm, e1_hbm, pred_out,
              i0_w, i1_w, a_v, b_v, out_v, sem):
    c = lax.axis_index("c")
    s = lax.axis_index("s")
    w = s * NC + c
    base = w * ELW
    lanes = lax.iota(jnp.int32, 16)

    def burst(t, carry):
        off = base + t * 128
        pltpu.sync_copy(e0_hbm.at[pl.ds(off, 128)], i0_w)
        pltpu.sync_copy(e1_hbm.at[pl.ds(off, 128)], i1_w)
        d0 = pltpu.async_copy(h_hbm.at[i0_w], a_v, sem)
        d1 = pltpu.async_copy(h_hbm.at[i1_w], b_v, sem)
        d0.wait()
        d1.wait()

        def grp(g, carry2):
            res = jnp.zeros((16,), jnp.float32)
            for r16 in range(16):
                r = g * 16 + r16
                acc = a_v[r, pl.ds(0, 16)] * b_v[r, pl.ds(0, 16)]
                for j in range(1, D // 16):
                    acc = acc + (a_v[r, pl.ds(j * 16, 16)]
                                 * b_v[r, pl.ds(j * 16, 16)])
                for k in (8, 4, 2, 1):
                    acc = acc + jnp.take(acc, lanes ^ k)
                res = jnp.where(lanes == r16, acc, res)
            out_v[pl.ds(g * 16, 16)] = res
            return carry2

        lax.fori_loop(0, 8, grp, 0)
        pltpu.sync_copy(out_v, pred_out.at[pl.ds(off, 128)])
        return carry

    lax.fori_loop(0, ELW // 128, burst, 0)


_cls_kernel = pl.kernel(
    _cls_body,
    out_type=jax.ShapeDtypeStruct((ELPAD,), jnp.float32),
    mesh=plsc.VectorSubcoreMesh(**_MESH),
    scratch_types=[
        pltpu.VMEM((128,), jnp.int32),
        pltpu.VMEM((128,), jnp.int32),
        pltpu.VMEM((128, D), jnp.float32),
        pltpu.VMEM((128, D), jnp.float32),
        pltpu.VMEM((128,), jnp.float32),
        pltpu.SemaphoreType.DMA,
    ],
)


def _tc_body(relu, agg_ref, cnt_ref, x_ref, wl_ref, wr_ref, bl_ref, out_ref):
    aggs = agg_ref[0] + agg_ref[1]
    cnt = cnt_ref[0, :, 0:1] + cnt_ref[1, :, 0:1]
    mean = aggs / jnp.maximum(cnt, 1.0)
    h = lax.dot_general(mean, wl_ref[...], (((1,), (1,)), ((), ())),
                        preferred_element_type=jnp.float32)
    h = h + bl_ref[...]
    h = h + lax.dot_general(x_ref[...], wr_ref[...], (((1,), (1,)), ((), ())),
                            preferred_element_type=jnp.float32)
    if relu:
        h = jnp.maximum(h, 0.0)
    out_ref[...] = h


def _tc_layer(relu, agg, cnt, x, wl, wr, bl):
    R = 1000
    grid = (N // R,)
    return pl.pallas_call(
        functools.partial(_tc_body, relu),
        grid=grid,
        in_specs=[
            pl.BlockSpec((NC, R, D), lambda i: (0, i, 0)),
            pl.BlockSpec((NC, R, D), lambda i: (0, i, 0)),
            pl.BlockSpec((R, D), lambda i: (i, 0)),
            pl.BlockSpec((D, D), lambda i: (0, 0)),
            pl.BlockSpec((D, D), lambda i: (0, 0)),
            pl.BlockSpec((1, D), lambda i: (0, 0)),
        ],
        out_specs=pl.BlockSpec((R, D), lambda i: (i, 0)),
        out_shape=jax.ShapeDtypeStruct((N, D), jnp.float32),
    )(agg, cnt, x, wl, wr, bl)


_seg_sum = _make_seg_sum(True)
_cnt_sum = _make_seg_sum(False)


def kernel(x, edge_index, edge_label_index, Wl1, bl1, Wr1, Wl2, bl2, Wr2):
    ei = edge_index.astype(jnp.int32)
    eli = edge_label_index.astype(jnp.int32)

    pad = EPAD - E
    src = jnp.concatenate([ei[0], jnp.zeros((pad,), jnp.int32)])
    dst = jnp.concatenate([ei[1], jnp.full((pad,), N, jnp.int32)])

    z128 = jnp.zeros((NR, D), jnp.float32)

    agg1 = _seg_sum(src, dst, x, z128).reshape(NC, NR, D)
    cnt = _cnt_sum(dst, z128).reshape(NC, NR, D)
    h1 = _tc_layer(True, agg1, cnt, x, Wl1, Wr1, bl1.reshape(1, D))
    agg2 = _seg_sum(src, dst, h1, z128).reshape(NC, NR, D)
    h2 = _tc_layer(False, agg2, cnt, h1, Wl2, Wr2, bl2.reshape(1, D))

    lpad = ELPAD - EL
    e0 = jnp.concatenate([eli[0], jnp.zeros((lpad,), jnp.int32)])
    e1 = jnp.concatenate([eli[1], jnp.zeros((lpad,), jnp.int32)])
    pred = _cls_kernel(h2, e0, e1)
    return pred[:EL]
